# Initial kernel scaffold; baseline (speedup 1.0000x reference)
import jax
import jax.numpy as jnp
from jax.experimental import pallas as pl


def kernel(boxes, scores):
    raise NotImplementedError("write your pallas kernel here")


# speedup vs baseline: 12.5667x; 12.5667x over previous
import jax
import jax.numpy as jnp
from jax.experimental import pallas as pl
from jax.experimental.pallas import tpu as pltpu

_PRE = 2000
_POST = 1000
_T = 0.7
_NB = 16
_BS = 128
_PAD = _NB * _BS


def _nms_kernel(x0r_ref, y0r_ref, x1r_ref, y1r_ref,
                cx0_ref, cy0_ref, cx1_ref, cy1_ref,
                keep_ref, sup_ref):
    lane = jax.lax.broadcasted_iota(jnp.int32, (1, _BS), 1)
    keep_ref[...] = jnp.ones((_NB, _BS), jnp.float32)

    def tile_iou(b, c):
        bx0 = cx0_ref[pl.ds(b, 1)].reshape(_BS, _BS)
        by0 = cy0_ref[pl.ds(b, 1)].reshape(_BS, _BS)
        bx1 = cx1_ref[pl.ds(b, 1)].reshape(_BS, _BS)
        by1 = cy1_ref[pl.ds(b, 1)].reshape(_BS, _BS)
        rx0 = x0r_ref[pl.ds(c, 1), :]
        ry0 = y0r_ref[pl.ds(c, 1), :]
        rx1 = x1r_ref[pl.ds(c, 1), :]
        ry1 = y1r_ref[pl.ds(c, 1), :]
        ba = (bx1 - bx0) * (by1 - by0)
        ra = (rx1 - rx0) * (ry1 - ry0)
        iw = jnp.maximum(jnp.minimum(bx1, rx1) - jnp.maximum(bx0, rx0), 0.0)
        ih = jnp.maximum(jnp.minimum(by1, ry1) - jnp.maximum(by0, ry0), 0.0)
        inter = iw * ih
        return inter / (ba + ra - inter + 1e-9)

    def outer(b, carry):
        sup_ref[...] = jnp.where(tile_iou(b, b) > _T, 1.0, 0.0)
        kb0 = keep_ref[pl.ds(b, 1), :]

        def inner(i, kb):
            row = sup_ref[pl.ds(i, 1), :]
            ki = jnp.sum(jnp.where(lane == i, kb, 0.0))
            sup = (row > 0.0) & (lane > i) & (ki > 0.0)
            return jnp.where(sup, 0.0, kb)

        kb = jax.lax.fori_loop(0, _BS, inner, kb0)
        keep_ref[pl.ds(b, 1), :] = kb
        kb8 = jnp.broadcast_to(kb, (8, _BS))

        def cross(c, carry):
            supf = jnp.where(tile_iou(b, c) > _T, 1.0, 0.0)
            cnt = jnp.dot(kb8, supf,
                          preferred_element_type=jnp.float32)[0:1, :]
            kc = keep_ref[pl.ds(c, 1), :]
            keep_ref[pl.ds(c, 1), :] = jnp.where(
                (cnt > 0.0) & (c > b), 0.0, kc)
            return carry

        return jax.lax.fori_loop(0, _NB, cross, carry)

    jax.lax.fori_loop(0, _NB, outer, 0)


def _run_nms(boxes_top):
    bp = jnp.concatenate(
        [boxes_top, jnp.zeros((_PAD - _PRE, 4), boxes_top.dtype)], axis=0)
    rows = [bp[:, k].reshape(_NB, _BS) for k in range(4)]
    cols = [jnp.broadcast_to(r[:, :, None], (_NB, _BS, _BS)) for r in rows]
    keep = pl.pallas_call(
        _nms_kernel,
        out_shape=jax.ShapeDtypeStruct((_NB, _BS), jnp.float32),
        scratch_shapes=[pltpu.VMEM((_BS, _BS), jnp.float32)],
    )(*rows, *cols)
    return keep.reshape(_PAD)[:_PRE] > 0.0


def kernel(boxes, scores):
    scores_top, idx = jax.lax.top_k(scores, _PRE)
    boxes_top = jnp.take(boxes, idx, axis=0)
    keep = _run_nms(boxes_top)
    w = boxes_top[:, 2] - boxes_top[:, 0]
    h = boxes_top[:, 3] - boxes_top[:, 1]
    valid = (w > 0.0) & (h > 0.0)
    masked = jnp.where(keep & valid, scores_top, -1e9)
    final_scores, order = jax.lax.top_k(masked, _POST)
    final_boxes = jnp.take(boxes_top, order, axis=0)
    return jnp.concatenate([final_boxes, final_scores[:, None]], axis=-1)


# speedup vs baseline: 12.7577x; 1.0152x over previous
import jax
import jax.numpy as jnp
from jax.experimental import pallas as pl
from jax.experimental.pallas import tpu as pltpu

_PRE = 2000
_POST = 1000
_T = 0.7
_NB = 16
_BS = 128
_PAD = _NB * _BS


def _nms_kernel(x0r_ref, y0r_ref, x1r_ref, y1r_ref,
                cx0_ref, cy0_ref, cx1_ref, cy1_ref, sc_ref,
                out_ref, keep_ref, sup_ref):
    lane = jax.lax.broadcasted_iota(jnp.int32, (1, _BS), 1)
    keep_ref[...] = jnp.ones((_NB, _BS), jnp.float32)

    def tile_iou(b, c):
        bx0 = cx0_ref[pl.ds(b, 1)].reshape(_BS, _BS)
        by0 = cy0_ref[pl.ds(b, 1)].reshape(_BS, _BS)
        bx1 = cx1_ref[pl.ds(b, 1)].reshape(_BS, _BS)
        by1 = cy1_ref[pl.ds(b, 1)].reshape(_BS, _BS)
        rx0 = x0r_ref[pl.ds(c, 1), :]
        ry0 = y0r_ref[pl.ds(c, 1), :]
        rx1 = x1r_ref[pl.ds(c, 1), :]
        ry1 = y1r_ref[pl.ds(c, 1), :]
        ba = (bx1 - bx0) * (by1 - by0)
        ra = (rx1 - rx0) * (ry1 - ry0)
        iw = jnp.maximum(jnp.minimum(bx1, rx1) - jnp.maximum(bx0, rx0), 0.0)
        ih = jnp.maximum(jnp.minimum(by1, ry1) - jnp.maximum(by0, ry0), 0.0)
        inter = iw * ih
        return inter / (ba + ra - inter + 1e-9)

    def outer(b, carry):
        sup_ref[...] = jnp.where(tile_iou(b, b) > _T, 1.0, 0.0)
        kb0 = keep_ref[pl.ds(b, 1), :]

        def inner(i8, kb):
            chunk = sup_ref[pl.ds(i8 * 8, 8), :]
            for j in range(8):
                i = i8 * 8 + j
                row = chunk[j:j + 1, :]
                ki = jnp.sum(jnp.where(lane == i, kb, 0.0))
                sup = (row > 0.0) & (lane > i) & (ki > 0.0)
                kb = jnp.where(sup, 0.0, kb)
            return kb

        kb = jax.lax.fori_loop(0, _BS // 8, inner, kb0)
        keep_ref[pl.ds(b, 1), :] = kb
        kb8 = jnp.broadcast_to(kb, (8, _BS))

        def cross(c, carry):
            supf = jnp.where(tile_iou(b, c) > _T, 1.0, 0.0)
            cnt = jnp.dot(kb8, supf,
                          preferred_element_type=jnp.float32)[0:1, :]
            kc = keep_ref[pl.ds(c, 1), :]
            keep_ref[pl.ds(c, 1), :] = jnp.where(
                (cnt > 0.0) & (c > b), 0.0, kc)
            return carry

        return jax.lax.fori_loop(0, _NB, cross, carry)

    jax.lax.fori_loop(0, _NB, outer, 0)

    w = x1r_ref[...] - x0r_ref[...]
    h = y1r_ref[...] - y0r_ref[...]
    ok = (keep_ref[...] > 0.0) & (w > 0.0) & (h > 0.0)
    out_ref[...] = jnp.where(ok, sc_ref[...], -1e9)


def _run_nms(boxes_top, scores_top):
    bp = jnp.concatenate(
        [boxes_top, jnp.zeros((_PAD - _PRE, 4), boxes_top.dtype)], axis=0)
    sp = jnp.concatenate(
        [scores_top, jnp.zeros((_PAD - _PRE,), scores_top.dtype)], axis=0)
    rows = [bp[:, k].reshape(_NB, _BS) for k in range(4)]
    cols = [jnp.broadcast_to(r[:, :, None], (_NB, _BS, _BS)) for r in rows]
    masked, _ = pl.pallas_call(
        _nms_kernel,
        out_shape=(jax.ShapeDtypeStruct((_NB, _BS), jnp.float32),
                   jax.ShapeDtypeStruct((_NB, _BS), jnp.float32)),
        scratch_shapes=[pltpu.VMEM((_BS, _BS), jnp.float32)],
    )(*rows, *cols, sp.reshape(_NB, _BS))
    return masked.reshape(_PAD)[:_PRE]


def kernel(boxes, scores):
    scores_top, idx = jax.lax.top_k(scores, _PRE)
    boxes_top = jnp.take(boxes, idx, axis=0)
    masked = _run_nms(boxes_top, scores_top)
    final_scores, order = jax.lax.top_k(masked, _POST)
    final_boxes = jnp.take(boxes_top, order, axis=0)
    return jnp.concatenate([final_boxes, final_scores[:, None]], axis=-1)


# speedup vs baseline: 13.6591x; 1.0707x over previous
import jax
import jax.numpy as jnp
from jax.experimental import pallas as pl
from jax.experimental.pallas import tpu as pltpu

_PRE = 2000
_POST = 1000
_T = 0.7
_NB = 16
_BS = 128
_PAD = _NB * _BS


def _nms_kernel(x0r_ref, y0r_ref, x1r_ref, y1r_ref,
                cx0_ref, cy0_ref, cx1_ref, cy1_ref, sc_ref,
                out_ref, keep_ref, sup_ref):
    lane = jax.lax.broadcasted_iota(jnp.int32, (1, _BS), 1)
    keep_ref[...] = jnp.ones((_NB, _BS), jnp.float32)

    def outer(b, carry):
        bx0 = cx0_ref[pl.ds(b, 1)].reshape(_BS, _BS)
        by0 = cy0_ref[pl.ds(b, 1)].reshape(_BS, _BS)
        bx1 = cx1_ref[pl.ds(b, 1)].reshape(_BS, _BS)
        by1 = cy1_ref[pl.ds(b, 1)].reshape(_BS, _BS)
        ba = (bx1 - bx0) * (by1 - by0)

        def tile_iou(c):
            rx0 = x0r_ref[pl.ds(c, 1), :]
            ry0 = y0r_ref[pl.ds(c, 1), :]
            rx1 = x1r_ref[pl.ds(c, 1), :]
            ry1 = y1r_ref[pl.ds(c, 1), :]
            ra = (rx1 - rx0) * (ry1 - ry0)
            iw = jnp.maximum(
                jnp.minimum(bx1, rx1) - jnp.maximum(bx0, rx0), 0.0)
            ih = jnp.maximum(
                jnp.minimum(by1, ry1) - jnp.maximum(by0, ry0), 0.0)
            inter = iw * ih
            return inter / (ba + ra - inter + 1e-9)

        sup_ref[...] = jnp.where(tile_iou(b) > _T, 1.0, 0.0)
        kb0 = keep_ref[pl.ds(b, 1), :]

        def inner(i8, kb):
            chunk = sup_ref[pl.ds(i8 * 8, 8), :]
            for j in range(8):
                i = i8 * 8 + j
                row = chunk[j:j + 1, :]
                ki = jnp.sum(jnp.where(lane == i, kb, 0.0))
                sup = (row > 0.0) & (lane > i) & (ki > 0.0)
                kb = jnp.where(sup, 0.0, kb)
            return kb

        kb = jax.lax.fori_loop(0, _BS // 8, inner, kb0)
        keep_ref[pl.ds(b, 1), :] = kb
        kb8 = jnp.broadcast_to(kb, (8, _BS))

        def cross(c, carry):
            supf = jnp.where(tile_iou(c) > _T, 1.0, 0.0)
            cnt = jnp.dot(kb8, supf,
                          preferred_element_type=jnp.float32)[0:1, :]
            kc = keep_ref[pl.ds(c, 1), :]
            keep_ref[pl.ds(c, 1), :] = jnp.where(cnt > 0.0, 0.0, kc)
            return carry

        return jax.lax.fori_loop(b + 1, _NB, cross, carry)

    jax.lax.fori_loop(0, _NB, outer, 0)

    w = x1r_ref[...] - x0r_ref[...]
    h = y1r_ref[...] - y0r_ref[...]
    ok = (keep_ref[...] > 0.0) & (w > 0.0) & (h > 0.0)
    out_ref[...] = jnp.where(ok, sc_ref[...], -1e9)


def _run_nms(boxes_top, scores_top):
    bp = jnp.concatenate(
        [boxes_top, jnp.zeros((_PAD - _PRE, 4), boxes_top.dtype)], axis=0)
    sp = jnp.concatenate(
        [scores_top, jnp.zeros((_PAD - _PRE,), scores_top.dtype)], axis=0)
    rows = [bp[:, k].reshape(_NB, _BS) for k in range(4)]
    cols = [jnp.broadcast_to(r[:, :, None], (_NB, _BS, _BS)) for r in rows]
    masked, _ = pl.pallas_call(
        _nms_kernel,
        out_shape=(jax.ShapeDtypeStruct((_NB, _BS), jnp.float32),
                   jax.ShapeDtypeStruct((_NB, _BS), jnp.float32)),
        scratch_shapes=[pltpu.VMEM((_BS, _BS), jnp.float32)],
    )(*rows, *cols, sp.reshape(_NB, _BS))
    return masked.reshape(_PAD)[:_PRE]


def kernel(boxes, scores):
    scores_top, idx = jax.lax.top_k(scores, _PRE)
    boxes_top = jnp.take(boxes, idx, axis=0)
    masked = _run_nms(boxes_top, scores_top)
    final_scores, order = jax.lax.top_k(masked, _POST)
    final_boxes = jnp.take(boxes_top, order, axis=0)
    return jnp.concatenate([final_boxes, final_scores[:, None]], axis=-1)


# speedup vs baseline: 50.1740x; 3.6733x over previous
import jax
import jax.numpy as jnp
from jax.experimental import pallas as pl

_PRE = 2000
_POST = 1000
_T = 0.7
_NB = 16
_BS = 128
_PAD = _NB * _BS


def _nms_kernel(x0r_ref, y0r_ref, x1r_ref, y1r_ref,
                cx0_ref, cy0_ref, cx1_ref, cy1_ref, sc_ref,
                out_ref, keep_ref):
    keep_ref[...] = jnp.ones((_NB, _BS), jnp.float32)
    upper = (jax.lax.broadcasted_iota(jnp.int32, (_BS, _BS), 0)
             < jax.lax.broadcasted_iota(jnp.int32, (_BS, _BS), 1))

    def outer(b, carry):
        bx0 = cx0_ref[pl.ds(b, 1)].reshape(_BS, _BS)
        by0 = cy0_ref[pl.ds(b, 1)].reshape(_BS, _BS)
        bx1 = cx1_ref[pl.ds(b, 1)].reshape(_BS, _BS)
        by1 = cy1_ref[pl.ds(b, 1)].reshape(_BS, _BS)
        ba = (bx1 - bx0) * (by1 - by0)

        def tile_iou(c):
            rx0 = x0r_ref[pl.ds(c, 1), :]
            ry0 = y0r_ref[pl.ds(c, 1), :]
            rx1 = x1r_ref[pl.ds(c, 1), :]
            ry1 = y1r_ref[pl.ds(c, 1), :]
            ra = (rx1 - rx0) * (ry1 - ry0)
            iw = jnp.maximum(
                jnp.minimum(bx1, rx1) - jnp.maximum(bx0, rx0), 0.0)
            ih = jnp.maximum(
                jnp.minimum(by1, ry1) - jnp.maximum(by0, ry0), 0.0)
            inter = iw * ih
            return inter / (ba + ra - inter + 1e-9)

        supf_bb = jnp.where((tile_iou(b) > _T) & upper, 1.0, 0.0)
        kb0 = keep_ref[pl.ds(b, 1), :]

        def cond(st):
            k, prev = st
            return jnp.sum(jnp.abs(k - prev)) > 0.0

        def body(st):
            k, _ = st
            cnt = jnp.dot(jnp.broadcast_to(k, (8, _BS)), supf_bb,
                          preferred_element_type=jnp.float32)[0:1, :]
            return (jnp.where(cnt > 0.0, 0.0, kb0), k)

        kb, _ = jax.lax.while_loop(cond, body, (kb0, kb0 - 1.0))
        keep_ref[pl.ds(b, 1), :] = kb
        kb8 = jnp.broadcast_to(kb, (8, _BS))

        def cross(c, carry):
            supf = jnp.where(tile_iou(c) > _T, 1.0, 0.0)
            cnt = jnp.dot(kb8, supf,
                          preferred_element_type=jnp.float32)[0:1, :]
            kc = keep_ref[pl.ds(c, 1), :]
            keep_ref[pl.ds(c, 1), :] = jnp.where(cnt > 0.0, 0.0, kc)
            return carry

        return jax.lax.fori_loop(b + 1, _NB, cross, carry)

    jax.lax.fori_loop(0, _NB, outer, 0)

    w = x1r_ref[...] - x0r_ref[...]
    h = y1r_ref[...] - y0r_ref[...]
    ok = (keep_ref[...] > 0.0) & (w > 0.0) & (h > 0.0)
    out_ref[...] = jnp.where(ok, sc_ref[...], -1e9)


def _run_nms(boxes_top, scores_top):
    bp = jnp.concatenate(
        [boxes_top, jnp.zeros((_PAD - _PRE, 4), boxes_top.dtype)], axis=0)
    sp = jnp.concatenate(
        [scores_top, jnp.zeros((_PAD - _PRE,), scores_top.dtype)], axis=0)
    rows = [bp[:, k].reshape(_NB, _BS) for k in range(4)]
    cols = [jnp.broadcast_to(r[:, :, None], (_NB, _BS, _BS)) for r in rows]
    masked, _ = pl.pallas_call(
        _nms_kernel,
        out_shape=(jax.ShapeDtypeStruct((_NB, _BS), jnp.float32),
                   jax.ShapeDtypeStruct((_NB, _BS), jnp.float32)),
    )(*rows, *cols, sp.reshape(_NB, _BS))
    return masked.reshape(_PAD)[:_PRE]


def kernel(boxes, scores):
    scores_top, idx = jax.lax.top_k(scores, _PRE)
    boxes_top = jnp.take(boxes, idx, axis=0)
    masked = _run_nms(boxes_top, scores_top)
    final_scores, order = jax.lax.top_k(masked, _POST)
    final_boxes = jnp.take(boxes_top, order, axis=0)
    return jnp.concatenate([final_boxes, final_scores[:, None]], axis=-1)
